# baseline (speedup 1.0000x reference)
import functools

import jax
import jax.numpy as jnp
from jax import lax
from jax.experimental import pallas as pl
from jax.experimental.pallas import tpu as pltpu
from jax.experimental.pallas import tpu_sc as plsc

T = 2048
D = 1024
C = 1024
E = 8
K = 2

EP = 128
BB = 256
NB = 24
PT = NB * BB
NSLOT = K * T


def _router_pos_kernel(x_ref, gw_ref, gb_ref, pos_out, wgt_out, cnt_out):
    xb = x_ref[...]

    logits = jnp.concatenate([
        lax.dot_general(xb[i * 256:(i + 1) * 256], gw_ref[...],
                        (((1,), (0,)), ((), ())),
                        preferred_element_type=jnp.float32)
        for i in range(T // 256)], axis=0)
    logits = logits + gb_ref[0:1, :]
    lane = lax.broadcasted_iota(jnp.int32, (T, EP), 1)
    logits = jnp.where(lane < E, logits, -jnp.inf)
    l1 = jnp.max(logits, axis=-1, keepdims=True)
    i1 = jnp.min(jnp.where(logits == l1, lane, 999),
                 axis=-1, keepdims=True)
    logits2 = jnp.where(lane == i1, -jnp.inf, logits)
    l2 = jnp.max(logits2, axis=-1, keepdims=True)
    i2 = jnp.min(jnp.where(logits2 == l2, lane, 999),
                 axis=-1, keepdims=True)
    p = jnp.exp(logits - l1)
    p = jnp.where(lane < E, p, 0.0)
    probs = p / jnp.sum(p, axis=-1, keepdims=True)
    m1 = jnp.sum(jnp.where(lane == i1, probs, 0.0), axis=-1, keepdims=True)
    m2 = jnp.sum(jnp.where(lane == i2, probs, 0.0), axis=-1, keepdims=True)
    denom = m1 + m2
    w1 = m1 / denom
    w2 = m2 / denom

    ohf = jnp.where(lane < E, jnp.where(lane == i1, 1.0, 0.0),
                    jnp.where(lane < 2 * E,
                              jnp.where(lane - E == i2, 1.0, 0.0),
                              0.0))
    ohb = ohf.astype(jnp.bfloat16)

    li = lax.broadcasted_iota(jnp.int32, (BB, BB), 0)
    lj = lax.broadcasted_iota(jnp.int32, (BB, BB), 1)
    Ltri = jnp.where(li >= lj, 1.0, 0.0).astype(jnp.bfloat16)
    ranks = []
    for b in range(T // BB):
        rb = lax.dot_general(
            Ltri, ohb[b * BB:(b + 1) * BB, :], (((1,), (0,)), ((), ())),
            preferred_element_type=jnp.float32)
        ranks.append(rb)
    S = jnp.concatenate([rb[BB - 1:BB, :] for rb in ranks], axis=0)
    s8i = lax.broadcasted_iota(jnp.int32, (8, 8), 0)
    s8j = lax.broadcasted_iota(jnp.int32, (8, 8), 1)
    SL8 = jnp.where(s8i > s8j, 1.0, 0.0)
    Sx = lax.dot_general(SL8, S, (((1,), (0,)), ((), ())),
                         precision=lax.Precision.HIGHEST,
                         preferred_element_type=jnp.float32)
    rank = jnp.concatenate(
        [ranks[b] + Sx[b:b + 1, :] for b in range(T // BB)], axis=0)

    TT = rank[T - 1:T, :]

    ia = lax.broadcasted_iota(jnp.int32, (EP, EP), 0)
    ib_ = lax.broadcasted_iota(jnp.int32, (EP, EP), 1)
    Mfold = jnp.where(
        (ib_ < E) & ((ia == ib_) | (ia == ib_ + E)), 1.0, 0.0)
    texp = lax.dot_general(TT, Mfold, (((1,), (0,)), ((), ())),
                           precision=lax.Precision.HIGHEST,
                           preferred_element_type=jnp.float32)
    pe = jnp.floor((texp + (BB - 1)) * (1.0 / BB))
    pe = pe * BB
    Uex = jnp.where((ia < ib_) & (ia < E) & (ib_ < E), 1.0, 0.0)
    base = lax.dot_general(pe, Uex, (((1,), (0,)), ((), ())),
                           precision=lax.Precision.HIGHEST,
                           preferred_element_type=jnp.float32)
    Mb = jnp.where(((ib_ < E) & (ia == ib_))
                   | ((ib_ >= E) & (ib_ < 2 * E) & (ia == ib_ - E)), 1.0, 0.0)
    baseC = lax.dot_general(base, Mb, (((1,), (0,)), ((), ())),
                            precision=lax.Precision.HIGHEST,
                            preferred_element_type=jnp.float32)
    Mk = jnp.where((ib_ >= E) & (ib_ < 2 * E) & (ia == ib_ - E), 1.0, 0.0)
    koff = lax.dot_general(TT, Mk, (((1,), (0,)), ((), ())),
                           precision=lax.Precision.HIGHEST,
                           preferred_element_type=jnp.float32)

    posmat = (baseC + koff) + (rank - 1.0)
    pfull = posmat * ohf
    p0 = jnp.sum(jnp.where(lane < E, pfull, 0.0), axis=1, keepdims=True)
    p1 = jnp.sum(jnp.where(lane >= E, pfull, 0.0), axis=1, keepdims=True)

    pos_out[...] = jnp.where(lane == 0, p0,
                             jnp.where(lane == 1, p1, 0.0)).astype(jnp.int32)
    wgt_out[...] = jnp.where(lane == 0, w1,
                             jnp.where(lane == 1, w2, 0.0))
    cnt_out[...] = jnp.broadcast_to(texp, (8, EP)).astype(jnp.int32)


def _scatter_kernel(x_hbm, pos_hbm, wsl_hbm, xg_hbm, wrow_hbm,
                    idx_v, buf_v, wbuf_v, sem):
    wid = lax.axis_index("s") * 2 + lax.axis_index("c")
    for ch in range(2):
        s0 = wid * 128 + ch * 64
        t0 = lax.rem(s0, T)
        pltpu.sync_copy(pos_hbm.at[pl.ds(s0, 64)], idx_v)
        pltpu.sync_copy(x_hbm.at[pl.ds(t0, 64)], buf_v)
        pltpu.sync_copy(wsl_hbm.at[pl.ds(s0, 64)], wbuf_v)
        pltpu.async_copy(buf_v, xg_hbm.at[idx_v], sem).wait()
        pltpu.async_copy(wbuf_v, wrow_hbm.at[idx_v], sem).wait()


def _gemm_kernel(be_ref, nb_ref, xg_ref, wr_ref, w1_ref, b1_ref,
                 w2_ref, b2_ref, yg_ref):
    xb16 = xg_ref[...].astype(jnp.bfloat16)
    h = lax.dot_general(
        xb16, w1_ref[0], (((1,), (0,)), ((), ())),
        preferred_element_type=jnp.float32)
    h = jnp.maximum(h + b1_ref[0], 0.0).astype(jnp.bfloat16)
    y = lax.dot_general(
        h, w2_ref[0], (((1,), (0,)), ((), ())),
        preferred_element_type=jnp.float32)
    y = y + b2_ref[0]
    yg_ref[...] = y * wr_ref[...][:, 0:1]


def _combine_kernel(yg_hbm, p0_hbm, p1_hbm, out_hbm,
                    i0_v, i1_v, r0_v, r1_v, ob_v, sem):
    wid = lax.axis_index("s") * 2 + lax.axis_index("c")

    def chunk(ch, carry):
        t0 = wid * 64 + ch * 16
        pltpu.sync_copy(p0_hbm.at[pl.ds(t0, 16)], i0_v)
        pltpu.sync_copy(p1_hbm.at[pl.ds(t0, 16)], i1_v)
        pltpu.async_copy(yg_hbm.at[i0_v], r0_v, sem).wait()
        pltpu.async_copy(yg_hbm.at[i1_v], r1_v, sem).wait()
        for t in range(16):
            for c in range(C // 16):
                sl = pl.ds(c * 16, 16)
                ob_v[t, sl] = r0_v[t, sl] + r1_v[t, sl]
        pltpu.sync_copy(ob_v, out_hbm.at[pl.ds(t0, 16)])
        return carry

    lax.fori_loop(0, 4, chunk, 0)


@functools.partial(jax.jit)
def kernel(x, gate_W, gate_b, W1, b1, W2, b2):
    gw_pad = jnp.zeros((D, EP), jnp.float32).at[:, :E].set(gate_W)
    gb_pad = jnp.zeros((8, EP), jnp.float32).at[:, :E].set(gate_b[None, :])
    w1_16 = W1.astype(jnp.bfloat16)
    w2_16 = W2.astype(jnp.bfloat16)
    b1r = b1.reshape(E, 1, D)
    b2r = b2.reshape(E, 1, C)

    pos_o, wgt_o, cnt_o = pl.pallas_call(
        _router_pos_kernel,
        grid=(1,),
        in_specs=[
            pl.BlockSpec((T, D), lambda i: (0, 0)),
            pl.BlockSpec((D, EP), lambda i: (0, 0)),
            pl.BlockSpec((8, EP), lambda i: (0, 0)),
        ],
        out_specs=[
            pl.BlockSpec((T, EP), lambda i: (0, 0)),
            pl.BlockSpec((T, EP), lambda i: (0, 0)),
            pl.BlockSpec((8, EP), lambda i: (0, 0)),
        ],
        out_shape=[
            jax.ShapeDtypeStruct((T, EP), jnp.int32),
            jax.ShapeDtypeStruct((T, EP), jnp.float32),
            jax.ShapeDtypeStruct((8, EP), jnp.int32),
        ],
    )(x, gw_pad, gb_pad)

    p0tok = pos_o[:, 0]
    p1tok = pos_o[:, 1]
    pos_slots = jnp.concatenate([p0tok, p1tok])
    wslots = jnp.concatenate([wgt_o[:, 0], wgt_o[:, 1]])
    wslots16 = jnp.broadcast_to(wslots[:, None], (NSLOT, 128))
    counts = cnt_o[0, :E]

    pe = ((counts + (BB - 1)) // BB) * BB
    ends = jnp.cumsum(pe)
    nb = (ends[E - 1] // BB).astype(jnp.int32)
    ib = jnp.arange(NB, dtype=jnp.int32) * BB
    be = jnp.minimum(
        jnp.sum((ib[:, None] >= ends[None, :]).astype(jnp.int32), axis=1),
        E - 1).astype(jnp.int32)
    nb_arr = nb[None]

    mesh = plsc.VectorSubcoreMesh(core_axis_name="c", subcore_axis_name="s")

    DEBUG_B = False
    if DEBUG_B:
        toks = jnp.concatenate([jnp.arange(T), jnp.arange(T)])
        xg = jnp.zeros((PT, D), jnp.float32).at[pos_slots].set(x[toks])
        wrow = jnp.zeros((PT, 128), jnp.float32).at[pos_slots].set(
            jnp.broadcast_to(wslots[:, None], (NSLOT, 128)))
    else:
        xg, wrow = pl.kernel(
        _scatter_kernel,
        mesh=mesh,
        out_type=[
            jax.ShapeDtypeStruct((PT, D), jnp.float32),
            jax.ShapeDtypeStruct((PT, 128), jnp.float32),
        ],
        scratch_types=[
            pltpu.VMEM((64,), jnp.int32),
            pltpu.VMEM((64, D), jnp.float32),
            pltpu.VMEM((64, 128), jnp.float32),
            pltpu.SemaphoreType.DMA,
        ],
        )(x, pos_slots, wslots16)

    DEBUG_C = False
    if DEBUG_C:
        xgb = xg.reshape(NB, BB, D).astype(jnp.bfloat16)
        hh = jnp.maximum(
            jnp.einsum('bnd,bdh->bnh', xgb, w1_16[be],
                       preferred_element_type=jnp.float32)
            + b1[be][:, None, :], 0.0).astype(jnp.bfloat16)
        yy = jnp.einsum('bnh,bhc->bnc', hh, w2_16[be],
                        preferred_element_type=jnp.float32) + b2[be][:, None, :]
        yg = (yy * wrow.reshape(NB, BB, 128)[:, :, 0:1]).reshape(PT, C)
    else:
        yg = pl.pallas_call(
            _gemm_kernel,
            grid_spec=pltpu.PrefetchScalarGridSpec(
            num_scalar_prefetch=2,
            grid=(NB,),
            in_specs=[
                pl.BlockSpec((BB, D), lambda i, be, nb: (i, 0)),
                pl.BlockSpec((BB, 128), lambda i, be, nb: (i, 0)),
                pl.BlockSpec((1, D, D), lambda i, be, nb: (be[i], 0, 0)),
                pl.BlockSpec((1, 1, D), lambda i, be, nb: (be[i], 0, 0)),
                pl.BlockSpec((1, D, C), lambda i, be, nb: (be[i], 0, 0)),
                pl.BlockSpec((1, 1, C), lambda i, be, nb: (be[i], 0, 0)),
            ],
            out_specs=pl.BlockSpec((BB, C), lambda i, be, nb: (i, 0)),
        ),
        out_shape=jax.ShapeDtypeStruct((PT, C), jnp.float32),
    )(be, nb_arr, xg, wrow, w1_16, b1r, w2_16, b2r)

    DEBUG_D = False
    if DEBUG_D:
        out = yg[p0tok] + yg[p1tok]
    else:
        out = pl.kernel(
            _combine_kernel,
            mesh=mesh,
            out_type=jax.ShapeDtypeStruct((T, C), jnp.float32),
            scratch_types=[
                pltpu.VMEM((16,), jnp.int32),
                pltpu.VMEM((16,), jnp.int32),
                pltpu.VMEM((16, C), jnp.float32),
                pltpu.VMEM((16, C), jnp.float32),
                pltpu.VMEM((16, C), jnp.float32),
                pltpu.SemaphoreType.DMA,
            ],
        )(yg, p0tok, p1tok)

    return out


# speedup vs baseline: 1.0467x; 1.0467x over previous
import functools

import jax
import jax.numpy as jnp
from jax import lax
from jax.experimental import pallas as pl
from jax.experimental.pallas import tpu as pltpu
from jax.experimental.pallas import tpu_sc as plsc

T = 2048
D = 1024
C = 1024
E = 8
K = 2

EP = 128
BB = 256
NB = 24
PT = NB * BB
NSLOT = K * T


def _router_pos_kernel(x_ref, gw_ref, gb_ref, pos_out, wgt_out, cnt_out):
    xb = x_ref[...]

    logits = jnp.concatenate([
        lax.dot_general(xb[i * 256:(i + 1) * 256], gw_ref[...],
                        (((1,), (0,)), ((), ())),
                        preferred_element_type=jnp.float32)
        for i in range(T // 256)], axis=0)
    logits = logits + gb_ref[0:1, :]
    lane = lax.broadcasted_iota(jnp.int32, (T, EP), 1)
    logits = jnp.where(lane < E, logits, -jnp.inf)
    l1 = jnp.max(logits, axis=-1, keepdims=True)
    i1 = jnp.min(jnp.where(logits == l1, lane, 999),
                 axis=-1, keepdims=True)
    logits2 = jnp.where(lane == i1, -jnp.inf, logits)
    l2 = jnp.max(logits2, axis=-1, keepdims=True)
    i2 = jnp.min(jnp.where(logits2 == l2, lane, 999),
                 axis=-1, keepdims=True)
    p = jnp.exp(logits - l1)
    p = jnp.where(lane < E, p, 0.0)
    probs = p / jnp.sum(p, axis=-1, keepdims=True)
    m1 = jnp.sum(jnp.where(lane == i1, probs, 0.0), axis=-1, keepdims=True)
    m2 = jnp.sum(jnp.where(lane == i2, probs, 0.0), axis=-1, keepdims=True)
    denom = m1 + m2
    w1 = m1 / denom
    w2 = m2 / denom

    ohf = jnp.where(lane < E, jnp.where(lane == i1, 1.0, 0.0),
                    jnp.where(lane < 2 * E,
                              jnp.where(lane - E == i2, 1.0, 0.0),
                              0.0))
    ohb = ohf.astype(jnp.bfloat16)

    li = lax.broadcasted_iota(jnp.int32, (BB, BB), 0)
    lj = lax.broadcasted_iota(jnp.int32, (BB, BB), 1)
    Ltri = jnp.where(li >= lj, 1.0, 0.0).astype(jnp.bfloat16)
    ranks = []
    for b in range(T // BB):
        rb = lax.dot_general(
            Ltri, ohb[b * BB:(b + 1) * BB, :], (((1,), (0,)), ((), ())),
            preferred_element_type=jnp.float32)
        ranks.append(rb)
    S = jnp.concatenate([rb[BB - 1:BB, :] for rb in ranks], axis=0)
    s8i = lax.broadcasted_iota(jnp.int32, (8, 8), 0)
    s8j = lax.broadcasted_iota(jnp.int32, (8, 8), 1)
    SL8 = jnp.where(s8i > s8j, 1.0, 0.0)
    Sx = lax.dot_general(SL8, S, (((1,), (0,)), ((), ())),
                         precision=lax.Precision.HIGHEST,
                         preferred_element_type=jnp.float32)
    rank = jnp.concatenate(
        [ranks[b] + Sx[b:b + 1, :] for b in range(T // BB)], axis=0)

    TT = rank[T - 1:T, :]

    ia = lax.broadcasted_iota(jnp.int32, (EP, EP), 0)
    ib_ = lax.broadcasted_iota(jnp.int32, (EP, EP), 1)
    Mfold = jnp.where(
        (ib_ < E) & ((ia == ib_) | (ia == ib_ + E)), 1.0, 0.0)
    texp = lax.dot_general(TT, Mfold, (((1,), (0,)), ((), ())),
                           precision=lax.Precision.HIGHEST,
                           preferred_element_type=jnp.float32)
    pe = jnp.floor((texp + (BB - 1)) * (1.0 / BB))
    pe = pe * BB
    Uex = jnp.where((ia < ib_) & (ia < E) & (ib_ < E), 1.0, 0.0)
    base = lax.dot_general(pe, Uex, (((1,), (0,)), ((), ())),
                           precision=lax.Precision.HIGHEST,
                           preferred_element_type=jnp.float32)
    Mb = jnp.where(((ib_ < E) & (ia == ib_))
                   | ((ib_ >= E) & (ib_ < 2 * E) & (ia == ib_ - E)), 1.0, 0.0)
    baseC = lax.dot_general(base, Mb, (((1,), (0,)), ((), ())),
                            precision=lax.Precision.HIGHEST,
                            preferred_element_type=jnp.float32)
    Mk = jnp.where((ib_ >= E) & (ib_ < 2 * E) & (ia == ib_ - E), 1.0, 0.0)
    koff = lax.dot_general(TT, Mk, (((1,), (0,)), ((), ())),
                           precision=lax.Precision.HIGHEST,
                           preferred_element_type=jnp.float32)

    posmat = (baseC + koff) + (rank - 1.0)
    pfull = posmat * ohf
    p0 = jnp.sum(jnp.where(lane < E, pfull, 0.0), axis=1, keepdims=True)
    p1 = jnp.sum(jnp.where(lane >= E, pfull, 0.0), axis=1, keepdims=True)

    pos_out[...] = jnp.where(lane == 0, p0,
                             jnp.where(lane == 1, p1, 0.0)).astype(jnp.int32)
    wgt_out[...] = jnp.where(lane == 0, w1,
                             jnp.where(lane == 1, w2, 0.0))
    cnt_out[...] = jnp.broadcast_to(texp, (8, EP)).astype(jnp.int32)


def _scatter_kernel(x_hbm, pos_hbm, wsl_hbm, xg_hbm, wrow_hbm,
                    idx0_v, idx1_v, buf_v, wbuf0_v, wbuf1_v, sem):
    wid = lax.axis_index("s") * 2 + lax.axis_index("c")
    t0 = wid * 64
    pltpu.sync_copy(pos_hbm.at[pl.ds(t0, 64)], idx0_v)
    pltpu.sync_copy(pos_hbm.at[pl.ds(T + t0, 64)], idx1_v)
    pltpu.sync_copy(x_hbm.at[pl.ds(t0, 64)], buf_v)
    pltpu.sync_copy(wsl_hbm.at[pl.ds(t0, 64)], wbuf0_v)
    pltpu.sync_copy(wsl_hbm.at[pl.ds(T + t0, 64)], wbuf1_v)
    c0 = pltpu.async_copy(buf_v, xg_hbm.at[idx0_v], sem)
    c1 = pltpu.async_copy(buf_v, xg_hbm.at[idx1_v], sem)
    c2 = pltpu.async_copy(wbuf0_v, wrow_hbm.at[idx0_v], sem)
    c3 = pltpu.async_copy(wbuf1_v, wrow_hbm.at[idx1_v], sem)
    c0.wait()
    c1.wait()
    c2.wait()
    c3.wait()


def _gemm_kernel(be_ref, nb_ref, xg_ref, wr_ref, w1_ref, b1_ref,
                 w2_ref, b2_ref, yg_ref):
    xb16 = xg_ref[...].astype(jnp.bfloat16)
    h = lax.dot_general(
        xb16, w1_ref[0], (((1,), (0,)), ((), ())),
        preferred_element_type=jnp.float32)
    h = jnp.maximum(h + b1_ref[0], 0.0).astype(jnp.bfloat16)
    y = lax.dot_general(
        h, w2_ref[0], (((1,), (0,)), ((), ())),
        preferred_element_type=jnp.float32)
    y = y + b2_ref[0]
    yg_ref[...] = y * wr_ref[...][:, 0:1]


def _combine_kernel(yg_hbm, p0_hbm, p1_hbm, out_hbm,
                    i0_v, i1_v, r0_v, r1_v, ob_v, sem):
    wid = lax.axis_index("s") * 2 + lax.axis_index("c")
    base = wid * 64
    pltpu.sync_copy(p0_hbm.at[pl.ds(base, 64)], i0_v)
    pltpu.sync_copy(p1_hbm.at[pl.ds(base, 64)], i1_v)

    def chunk(ch, carry):
        t0 = base + ch * 16
        g0 = pltpu.async_copy(yg_hbm.at[i0_v.at[pl.ds(ch * 16, 16)]],
                              r0_v, sem)
        g1 = pltpu.async_copy(yg_hbm.at[i1_v.at[pl.ds(ch * 16, 16)]],
                              r1_v, sem)
        g0.wait()
        g1.wait()
        for t in range(16):
            for c in range(C // 16):
                sl = pl.ds(c * 16, 16)
                ob_v[t, sl] = r0_v[t, sl] + r1_v[t, sl]
        pltpu.sync_copy(ob_v, out_hbm.at[pl.ds(t0, 16)])
        return carry

    lax.fori_loop(0, 4, chunk, 0)


@functools.partial(jax.jit)
def kernel(x, gate_W, gate_b, W1, b1, W2, b2):
    gw_pad = jnp.zeros((D, EP), jnp.float32).at[:, :E].set(gate_W)
    gb_pad = jnp.zeros((8, EP), jnp.float32).at[:, :E].set(gate_b[None, :])
    w1_16 = W1.astype(jnp.bfloat16)
    w2_16 = W2.astype(jnp.bfloat16)
    b1r = b1.reshape(E, 1, D)
    b2r = b2.reshape(E, 1, C)

    pos_o, wgt_o, cnt_o = pl.pallas_call(
        _router_pos_kernel,
        grid=(1,),
        in_specs=[
            pl.BlockSpec((T, D), lambda i: (0, 0)),
            pl.BlockSpec((D, EP), lambda i: (0, 0)),
            pl.BlockSpec((8, EP), lambda i: (0, 0)),
        ],
        out_specs=[
            pl.BlockSpec((T, EP), lambda i: (0, 0)),
            pl.BlockSpec((T, EP), lambda i: (0, 0)),
            pl.BlockSpec((8, EP), lambda i: (0, 0)),
        ],
        out_shape=[
            jax.ShapeDtypeStruct((T, EP), jnp.int32),
            jax.ShapeDtypeStruct((T, EP), jnp.float32),
            jax.ShapeDtypeStruct((8, EP), jnp.int32),
        ],
    )(x, gw_pad, gb_pad)

    p0tok = pos_o[:, 0]
    p1tok = pos_o[:, 1]
    pos_slots = jnp.concatenate([p0tok, p1tok])
    wslots = jnp.concatenate([wgt_o[:, 0], wgt_o[:, 1]])
    wslots16 = jnp.broadcast_to(wslots[:, None], (NSLOT, 128))
    counts = cnt_o[0, :E]

    pe = ((counts + (BB - 1)) // BB) * BB
    ends = jnp.cumsum(pe)
    nb = (ends[E - 1] // BB).astype(jnp.int32)
    ib = jnp.arange(NB, dtype=jnp.int32) * BB
    be = jnp.minimum(
        jnp.sum((ib[:, None] >= ends[None, :]).astype(jnp.int32), axis=1),
        E - 1).astype(jnp.int32)
    nb_arr = nb[None]

    mesh = plsc.VectorSubcoreMesh(core_axis_name="c", subcore_axis_name="s")

    DEBUG_B = False
    if DEBUG_B:
        toks = jnp.concatenate([jnp.arange(T), jnp.arange(T)])
        xg = jnp.zeros((PT, D), jnp.float32).at[pos_slots].set(x[toks])
        wrow = jnp.zeros((PT, 128), jnp.float32).at[pos_slots].set(
            jnp.broadcast_to(wslots[:, None], (NSLOT, 128)))
    else:
        xg, wrow = pl.kernel(
        _scatter_kernel,
        mesh=mesh,
        out_type=[
            jax.ShapeDtypeStruct((PT, D), jnp.float32),
            jax.ShapeDtypeStruct((PT, 128), jnp.float32),
        ],
        scratch_types=[
            pltpu.VMEM((64,), jnp.int32),
            pltpu.VMEM((64,), jnp.int32),
            pltpu.VMEM((64, D), jnp.float32),
            pltpu.VMEM((64, 128), jnp.float32),
            pltpu.VMEM((64, 128), jnp.float32),
            pltpu.SemaphoreType.DMA,
        ],
        )(x, pos_slots, wslots16)

    DEBUG_C = False
    if DEBUG_C:
        xgb = xg.reshape(NB, BB, D).astype(jnp.bfloat16)
        hh = jnp.maximum(
            jnp.einsum('bnd,bdh->bnh', xgb, w1_16[be],
                       preferred_element_type=jnp.float32)
            + b1[be][:, None, :], 0.0).astype(jnp.bfloat16)
        yy = jnp.einsum('bnh,bhc->bnc', hh, w2_16[be],
                        preferred_element_type=jnp.float32) + b2[be][:, None, :]
        yg = (yy * wrow.reshape(NB, BB, 128)[:, :, 0:1]).reshape(PT, C)
    else:
        yg = pl.pallas_call(
            _gemm_kernel,
            grid_spec=pltpu.PrefetchScalarGridSpec(
            num_scalar_prefetch=2,
            grid=(NB,),
            in_specs=[
                pl.BlockSpec((BB, D), lambda i, be, nb: (i, 0)),
                pl.BlockSpec((BB, 128), lambda i, be, nb: (i, 0)),
                pl.BlockSpec((1, D, D), lambda i, be, nb: (be[i], 0, 0)),
                pl.BlockSpec((1, 1, D), lambda i, be, nb: (be[i], 0, 0)),
                pl.BlockSpec((1, D, C), lambda i, be, nb: (be[i], 0, 0)),
                pl.BlockSpec((1, 1, C), lambda i, be, nb: (be[i], 0, 0)),
            ],
            out_specs=pl.BlockSpec((BB, C), lambda i, be, nb: (i, 0)),
        ),
        out_shape=jax.ShapeDtypeStruct((PT, C), jnp.float32),
    )(be, nb_arr, xg, wrow, w1_16, b1r, w2_16, b2r)

    DEBUG_D = False
    if DEBUG_D:
        out = yg[p0tok] + yg[p1tok]
    else:
        out = pl.kernel(
            _combine_kernel,
            mesh=mesh,
            out_type=jax.ShapeDtypeStruct((T, C), jnp.float32),
            scratch_types=[
                pltpu.VMEM((64,), jnp.int32),
                pltpu.VMEM((64,), jnp.int32),
                pltpu.VMEM((16, C), jnp.float32),
                pltpu.VMEM((16, C), jnp.float32),
                pltpu.VMEM((16, C), jnp.float32),
                pltpu.SemaphoreType.DMA,
            ],
        )(yg, p0tok, p1tok)

    return out


# speedup vs baseline: 1.0651x; 1.0175x over previous
import functools

import jax
import jax.numpy as jnp
from jax import lax
from jax.experimental import pallas as pl
from jax.experimental.pallas import tpu as pltpu
from jax.experimental.pallas import tpu_sc as plsc

T = 2048
D = 1024
C = 1024
E = 8
K = 2

EP = 128
BB = 256
NB = 24
PT = NB * BB
NSLOT = K * T


def _router_pos_kernel(x_ref, gw_ref, gb_ref, pos_out, wgt_out, cnt_out):
    xb = x_ref[...]

    logits = jnp.concatenate([
        lax.dot_general(xb[i * 256:(i + 1) * 256], gw_ref[...],
                        (((1,), (0,)), ((), ())),
                        preferred_element_type=jnp.float32)
        for i in range(T // 256)], axis=0)
    logits = logits + gb_ref[0:1, :]
    lane = lax.broadcasted_iota(jnp.int32, (T, EP), 1)
    logits = jnp.where(lane < E, logits, -jnp.inf)
    l1 = jnp.max(logits, axis=-1, keepdims=True)
    i1 = jnp.min(jnp.where(logits == l1, lane, 999),
                 axis=-1, keepdims=True)
    logits2 = jnp.where(lane == i1, -jnp.inf, logits)
    l2 = jnp.max(logits2, axis=-1, keepdims=True)
    i2 = jnp.min(jnp.where(logits2 == l2, lane, 999),
                 axis=-1, keepdims=True)
    p = jnp.exp(logits - l1)
    p = jnp.where(lane < E, p, 0.0)
    probs = p / jnp.sum(p, axis=-1, keepdims=True)
    m1 = jnp.sum(jnp.where(lane == i1, probs, 0.0), axis=-1, keepdims=True)
    m2 = jnp.sum(jnp.where(lane == i2, probs, 0.0), axis=-1, keepdims=True)
    denom = m1 + m2
    w1 = m1 / denom
    w2 = m2 / denom

    ohf = jnp.where(lane < E, jnp.where(lane == i1, 1.0, 0.0),
                    jnp.where(lane < 2 * E,
                              jnp.where(lane - E == i2, 1.0, 0.0),
                              0.0))
    ohb = ohf.astype(jnp.bfloat16)

    li = lax.broadcasted_iota(jnp.int32, (BB, BB), 0)
    lj = lax.broadcasted_iota(jnp.int32, (BB, BB), 1)
    Ltri = jnp.where(li >= lj, 1.0, 0.0).astype(jnp.bfloat16)
    ranks = []
    for b in range(T // BB):
        rb = lax.dot_general(
            Ltri, ohb[b * BB:(b + 1) * BB, :], (((1,), (0,)), ((), ())),
            preferred_element_type=jnp.float32)
        ranks.append(rb)
    S = jnp.concatenate([rb[BB - 1:BB, :] for rb in ranks], axis=0)
    s8i = lax.broadcasted_iota(jnp.int32, (8, 8), 0)
    s8j = lax.broadcasted_iota(jnp.int32, (8, 8), 1)
    SL8 = jnp.where(s8i > s8j, 1.0, 0.0)
    Sx = lax.dot_general(SL8, S, (((1,), (0,)), ((), ())),
                         precision=lax.Precision.HIGHEST,
                         preferred_element_type=jnp.float32)
    rank = jnp.concatenate(
        [ranks[b] + Sx[b:b + 1, :] for b in range(T // BB)], axis=0)

    TT = rank[T - 1:T, :]

    ia = lax.broadcasted_iota(jnp.int32, (EP, EP), 0)
    ib_ = lax.broadcasted_iota(jnp.int32, (EP, EP), 1)
    Mfold = jnp.where(
        (ib_ < E) & ((ia == ib_) | (ia == ib_ + E)), 1.0, 0.0)
    texp = lax.dot_general(TT, Mfold, (((1,), (0,)), ((), ())),
                           precision=lax.Precision.HIGHEST,
                           preferred_element_type=jnp.float32)
    pe = jnp.floor((texp + (BB - 1)) * (1.0 / BB))
    pe = pe * BB
    Uex = jnp.where((ia < ib_) & (ia < E) & (ib_ < E), 1.0, 0.0)
    base = lax.dot_general(pe, Uex, (((1,), (0,)), ((), ())),
                           precision=lax.Precision.HIGHEST,
                           preferred_element_type=jnp.float32)
    Mb = jnp.where(((ib_ < E) & (ia == ib_))
                   | ((ib_ >= E) & (ib_ < 2 * E) & (ia == ib_ - E)), 1.0, 0.0)
    baseC = lax.dot_general(base, Mb, (((1,), (0,)), ((), ())),
                            precision=lax.Precision.HIGHEST,
                            preferred_element_type=jnp.float32)
    Mk = jnp.where((ib_ >= E) & (ib_ < 2 * E) & (ia == ib_ - E), 1.0, 0.0)
    koff = lax.dot_general(TT, Mk, (((1,), (0,)), ((), ())),
                           precision=lax.Precision.HIGHEST,
                           preferred_element_type=jnp.float32)

    posmat = (baseC + koff) + (rank - 1.0)
    pfull = posmat * ohf
    p0 = jnp.sum(jnp.where(lane < E, pfull, 0.0), axis=1, keepdims=True)
    p1 = jnp.sum(jnp.where(lane >= E, pfull, 0.0), axis=1, keepdims=True)

    pos_out[...] = jnp.where(lane == 0, p0,
                             jnp.where(lane == 1, p1, 0.0)).astype(jnp.int32)
    wgt_out[...] = jnp.where(lane == 0, w1,
                             jnp.where(lane == 1, w2, 0.0))
    cnt_out[...] = jnp.broadcast_to(texp, (8, EP)).astype(jnp.int32)


def _scatter_kernel(x_hbm, pos_hbm, wsl_hbm, xg_hbm, wrow_hbm,
                    idx0_v, idx1_v, buf_v, wbuf0_v, wbuf1_v, sem):
    wid = lax.axis_index("s") * 2 + lax.axis_index("c")
    t0 = wid * 64
    pltpu.sync_copy(pos_hbm.at[pl.ds(t0, 64)], idx0_v)
    pltpu.sync_copy(pos_hbm.at[pl.ds(T + t0, 64)], idx1_v)
    pltpu.sync_copy(x_hbm.at[pl.ds(t0, 64)], buf_v)
    pltpu.sync_copy(wsl_hbm.at[pl.ds(t0, 64)], wbuf0_v)
    pltpu.sync_copy(wsl_hbm.at[pl.ds(T + t0, 64)], wbuf1_v)
    c0 = pltpu.async_copy(buf_v, xg_hbm.at[idx0_v], sem)
    c1 = pltpu.async_copy(buf_v, xg_hbm.at[idx1_v], sem)
    c2 = pltpu.async_copy(wbuf0_v, wrow_hbm.at[idx0_v], sem)
    c3 = pltpu.async_copy(wbuf1_v, wrow_hbm.at[idx1_v], sem)
    c0.wait()
    c1.wait()
    c2.wait()
    c3.wait()


def _gemm_kernel(be_ref, nb_ref, xg_ref, wr_ref, w1_ref, b1_ref,
                 w2_ref, b2_ref, yg_ref):
    i = pl.program_id(0)

    @pl.when(i < nb_ref[0])
    def _():
        e = be_ref[i]
        xb16 = xg_ref[...].astype(jnp.bfloat16)
        h = lax.dot_general(
            xb16, w1_ref[e], (((1,), (0,)), ((), ())),
            preferred_element_type=jnp.float32)
        h = jnp.maximum(h + b1_ref[e], 0.0).astype(jnp.bfloat16)
        y = lax.dot_general(
            h, w2_ref[e], (((1,), (0,)), ((), ())),
            preferred_element_type=jnp.float32)
        y = y + b2_ref[e]
        yg_ref[...] = y * wr_ref[...][:, 0:1]


def _combine_kernel(yg_hbm, p0_hbm, p1_hbm, out_hbm,
                    i0_v, i1_v, r0_v, r1_v, ob_v, sem):
    wid = lax.axis_index("s") * 2 + lax.axis_index("c")
    base = wid * 64
    pltpu.sync_copy(p0_hbm.at[pl.ds(base, 64)], i0_v)
    pltpu.sync_copy(p1_hbm.at[pl.ds(base, 64)], i1_v)

    def chunk(ch, carry):
        t0 = base + ch * 16
        g0 = pltpu.async_copy(yg_hbm.at[i0_v.at[pl.ds(ch * 16, 16)]],
                              r0_v, sem)
        g1 = pltpu.async_copy(yg_hbm.at[i1_v.at[pl.ds(ch * 16, 16)]],
                              r1_v, sem)
        g0.wait()
        g1.wait()
        for t in range(16):
            for c in range(C // 16):
                sl = pl.ds(c * 16, 16)
                ob_v[t, sl] = r0_v[t, sl] + r1_v[t, sl]
        pltpu.sync_copy(ob_v, out_hbm.at[pl.ds(t0, 16)])
        return carry

    lax.fori_loop(0, 4, chunk, 0)


@functools.partial(jax.jit)
def kernel(x, gate_W, gate_b, W1, b1, W2, b2):
    gw_pad = jnp.zeros((D, EP), jnp.float32).at[:, :E].set(gate_W)
    gb_pad = jnp.zeros((8, EP), jnp.float32).at[:, :E].set(gate_b[None, :])
    w1_16 = W1.astype(jnp.bfloat16)
    w2_16 = W2.astype(jnp.bfloat16)
    b1r = b1.reshape(E, 1, D)
    b2r = b2.reshape(E, 1, C)

    pos_o, wgt_o, cnt_o = pl.pallas_call(
        _router_pos_kernel,
        grid=(1,),
        in_specs=[
            pl.BlockSpec((T, D), lambda i: (0, 0)),
            pl.BlockSpec((D, EP), lambda i: (0, 0)),
            pl.BlockSpec((8, EP), lambda i: (0, 0)),
        ],
        out_specs=[
            pl.BlockSpec((T, EP), lambda i: (0, 0)),
            pl.BlockSpec((T, EP), lambda i: (0, 0)),
            pl.BlockSpec((8, EP), lambda i: (0, 0)),
        ],
        out_shape=[
            jax.ShapeDtypeStruct((T, EP), jnp.int32),
            jax.ShapeDtypeStruct((T, EP), jnp.float32),
            jax.ShapeDtypeStruct((8, EP), jnp.int32),
        ],
    )(x, gw_pad, gb_pad)

    p0tok = pos_o[:, 0]
    p1tok = pos_o[:, 1]
    pos_slots = jnp.concatenate([p0tok, p1tok])
    wslots = jnp.concatenate([wgt_o[:, 0], wgt_o[:, 1]])
    wslots16 = jnp.broadcast_to(wslots[:, None], (NSLOT, 128))
    counts = cnt_o[0, :E]

    pe = ((counts + (BB - 1)) // BB) * BB
    ends = jnp.cumsum(pe)
    nb = (ends[E - 1] // BB).astype(jnp.int32)
    ib = jnp.arange(NB, dtype=jnp.int32) * BB
    be = jnp.minimum(
        jnp.sum((ib[:, None] >= ends[None, :]).astype(jnp.int32), axis=1),
        E - 1).astype(jnp.int32)
    nb_arr = nb[None]

    mesh = plsc.VectorSubcoreMesh(core_axis_name="c", subcore_axis_name="s")

    DEBUG_B = False
    if DEBUG_B:
        toks = jnp.concatenate([jnp.arange(T), jnp.arange(T)])
        xg = jnp.zeros((PT, D), jnp.float32).at[pos_slots].set(x[toks])
        wrow = jnp.zeros((PT, 128), jnp.float32).at[pos_slots].set(
            jnp.broadcast_to(wslots[:, None], (NSLOT, 128)))
    else:
        xg, wrow = pl.kernel(
        _scatter_kernel,
        mesh=mesh,
        out_type=[
            jax.ShapeDtypeStruct((PT, D), jnp.float32),
            jax.ShapeDtypeStruct((PT, 128), jnp.float32),
        ],
        scratch_types=[
            pltpu.VMEM((64,), jnp.int32),
            pltpu.VMEM((64,), jnp.int32),
            pltpu.VMEM((64, D), jnp.float32),
            pltpu.VMEM((64, 128), jnp.float32),
            pltpu.VMEM((64, 128), jnp.float32),
            pltpu.SemaphoreType.DMA,
        ],
        )(x, pos_slots, wslots16)

    DEBUG_C = False
    if DEBUG_C:
        xgb = xg.reshape(NB, BB, D).astype(jnp.bfloat16)
        hh = jnp.maximum(
            jnp.einsum('bnd,bdh->bnh', xgb, w1_16[be],
                       preferred_element_type=jnp.float32)
            + b1[be][:, None, :], 0.0).astype(jnp.bfloat16)
        yy = jnp.einsum('bnh,bhc->bnc', hh, w2_16[be],
                        preferred_element_type=jnp.float32) + b2[be][:, None, :]
        yg = (yy * wrow.reshape(NB, BB, 128)[:, :, 0:1]).reshape(PT, C)
    else:
        yg = pl.pallas_call(
            _gemm_kernel,
            grid_spec=pltpu.PrefetchScalarGridSpec(
            num_scalar_prefetch=2,
            grid=(NB,),
            in_specs=[
                pl.BlockSpec((BB, D), lambda i, be, nb: (i, 0)),
                pl.BlockSpec((BB, 128), lambda i, be, nb: (i, 0)),
                pl.BlockSpec((E, D, D), lambda i, be, nb: (0, 0, 0)),
                pl.BlockSpec((E, 1, D), lambda i, be, nb: (0, 0, 0)),
                pl.BlockSpec((E, D, C), lambda i, be, nb: (0, 0, 0)),
                pl.BlockSpec((E, 1, C), lambda i, be, nb: (0, 0, 0)),
            ],
            out_specs=pl.BlockSpec((BB, C), lambda i, be, nb: (i, 0)),
        ),
        out_shape=jax.ShapeDtypeStruct((PT, C), jnp.float32),
    )(be, nb_arr, xg, wrow, w1_16, b1r, w2_16, b2r)

    DEBUG_D = False
    if DEBUG_D:
        out = yg[p0tok] + yg[p1tok]
    else:
        out = pl.kernel(
            _combine_kernel,
            mesh=mesh,
            out_type=jax.ShapeDtypeStruct((T, C), jnp.float32),
            scratch_types=[
                pltpu.VMEM((64,), jnp.int32),
                pltpu.VMEM((64,), jnp.int32),
                pltpu.VMEM((16, C), jnp.float32),
                pltpu.VMEM((16, C), jnp.float32),
                pltpu.VMEM((16, C), jnp.float32),
                pltpu.SemaphoreType.DMA,
            ],
        )(yg, p0tok, p1tok)

    return out
